# baseline (speedup 1.0000x reference)
import functools

import jax
import jax.numpy as jnp
from jax import lax
from jax.experimental import pallas as pl
from jax.experimental.pallas import tpu as pltpu
from jax.experimental.pallas import tpu_sc as plsc

N_OUT = 16384
SPAN = 1000000

_info = plsc.get_sparse_core_info()
_NC, _NS, _L = _info.num_cores, _info.num_subcores, _info.num_lanes
_NW = _NC * _NS
_PER_W = N_OUT // _NW
_VREGS = _PER_W // _L


def _rotl(x, r):
    return (x << jnp.uint32(r)) | (x >> jnp.uint32(32 - r))


def _threefry2x32(k0, k1, x0, x1):
    ks2 = k0 ^ k1 ^ jnp.uint32(0x1BD11BDA)
    ks = (k0, k1, ks2)
    rots = ((13, 15, 26, 6), (17, 29, 16, 24))
    x0 = x0 + k0
    x1 = x1 + k1
    for i in range(5):
        for r in rots[i % 2]:
            x0 = x0 + x1
            x1 = _rotl(x1, r)
            x1 = x1 ^ x0
        x0 = x0 + ks[(i + 1) % 3]
        x1 = x1 + ks[(i + 2) % 3] + jnp.uint32(i + 1)
    return x0, x1


def _mod_span(bits):
    q = (bits.astype(jnp.float32) * jnp.float32(1.0 / SPAN)).astype(jnp.int32)
    r = (bits - q.astype(jnp.uint32) * jnp.uint32(SPAN)).astype(jnp.int32)
    r = jnp.where(r < 0, r + jnp.int32(SPAN), r)
    r = jnp.where(r >= jnp.int32(SPAN), r - jnp.int32(SPAN), r)
    return r


def _sc_body(key_hbm, out_hbm, key_v, out_v):
    wid = lax.axis_index("s") * _NC + lax.axis_index("c")
    pltpu.sync_copy(key_hbm, key_v)
    kv = key_v[...]
    k0 = jnp.full((_L,), kv[0], jnp.uint32)
    k1 = jnp.full((_L,), kv[1], jnp.uint32)

    zero = jnp.zeros((_L,), jnp.uint32)
    c, d = _threefry2x32(k0, k1, zero, jnp.full((_L,), 1, jnp.uint32))

    base = (wid * _PER_W).astype(jnp.uint32)
    idx = lax.iota(jnp.uint32, _L)
    @plsc.parallel_loop(0, _VREGS)
    def _(j):
        cnt = idx + base + (j * _L).astype(jnp.uint32)
        y0, y1 = _threefry2x32(c, d, zero, cnt)
        out_v[pl.ds(j * _L, _L)] = _mod_span(y0 ^ y1)

    pltpu.sync_copy(out_v, out_hbm.at[pl.ds(wid * _PER_W, _PER_W)])


@jax.jit
def _sample(key_arr):
    mesh = plsc.VectorSubcoreMesh(
        core_axis_name="c", subcore_axis_name="s", num_cores=_NC)
    f = functools.partial(
        pl.kernel,
        mesh=mesh,
        out_type=jax.ShapeDtypeStruct((N_OUT,), jnp.int32),
        scratch_types=[
            pltpu.VMEM((_L,), jnp.uint32),
            pltpu.VMEM((_PER_W,), jnp.int32),
        ],
    )(_sc_body)
    return f(key_arr)


def kernel(key):
    kd = jax.random.key_data(key).astype(jnp.uint32)
    key_arr = jnp.zeros((_L,), jnp.uint32).at[:2].set(kd)
    return _sample(key_arr)


# speedup vs baseline: 1.0040x; 1.0040x over previous
import functools

import jax
import jax.numpy as jnp
from jax import lax
from jax.experimental import pallas as pl
from jax.experimental.pallas import tpu as pltpu
from jax.experimental.pallas import tpu_sc as plsc

N_OUT = 16384
SPAN = 1000000

_info = plsc.get_sparse_core_info()
_NC, _NS, _L = _info.num_cores, _info.num_subcores, _info.num_lanes
_NW = _NC * _NS
_PER_W = N_OUT // _NW
_VREGS = _PER_W // _L


def _rotl(x, r):
    return (x << jnp.uint32(r)) | (x >> jnp.uint32(32 - r))


def _threefry2x32(k0, k1, x0, x1):
    ks2 = k0 ^ k1 ^ jnp.uint32(0x1BD11BDA)
    ks = (k0, k1, ks2)
    rots = ((13, 15, 26, 6), (17, 29, 16, 24))
    x0 = x0 + k0
    x1 = x1 + k1
    for i in range(5):
        for r in rots[i % 2]:
            x0 = x0 + x1
            x1 = _rotl(x1, r)
            x1 = x1 ^ x0
        x0 = x0 + ks[(i + 1) % 3]
        x1 = x1 + ks[(i + 2) % 3] + jnp.uint32(i + 1)
    return x0, x1


def _mod_span(bits):
    q = (bits.astype(jnp.float32) * jnp.float32(1.0 / SPAN)).astype(jnp.int32)
    r = (bits - q.astype(jnp.uint32) * jnp.uint32(SPAN)).astype(jnp.int32)
    r = jnp.where(r < 0, r + jnp.int32(SPAN), r)
    r = jnp.where(r >= jnp.int32(SPAN), r - jnp.int32(SPAN), r)
    return r


def _sc_body(key_hbm, out_hbm, key_v, out_v):
    wid = lax.axis_index("s") * _NC + lax.axis_index("c")
    pltpu.sync_copy(key_hbm, key_v)
    kv = key_v[...]
    k0 = jnp.full((_L,), kv[0], jnp.uint32)
    k1 = jnp.full((_L,), kv[1], jnp.uint32)

    zero = jnp.zeros((_L,), jnp.uint32)
    c, d = _threefry2x32(k0, k1, zero, jnp.full((_L,), 1, jnp.uint32))

    base = (wid * _PER_W).astype(jnp.uint32)
    idx = lax.iota(jnp.uint32, _L)
    @plsc.parallel_loop(0, _VREGS, unroll=2)
    def _(j):
        cnt = idx + base + (j * _L).astype(jnp.uint32)
        y0, y1 = _threefry2x32(c, d, zero, cnt)
        out_v[pl.ds(j * _L, _L)] = _mod_span(y0 ^ y1)

    pltpu.sync_copy(out_v, out_hbm.at[pl.ds(wid * _PER_W, _PER_W)])


@jax.jit
def _sample(key_arr):
    mesh = plsc.VectorSubcoreMesh(
        core_axis_name="c", subcore_axis_name="s", num_cores=_NC)
    f = functools.partial(
        pl.kernel,
        mesh=mesh,
        out_type=jax.ShapeDtypeStruct((N_OUT,), jnp.int32),
        scratch_types=[
            pltpu.VMEM((_L,), jnp.uint32),
            pltpu.VMEM((_PER_W,), jnp.int32),
        ],
    )(_sc_body)
    return f(key_arr)


def kernel(key):
    kd = jax.random.key_data(key).astype(jnp.uint32)
    key_arr = jnp.zeros((_L,), jnp.uint32).at[:2].set(kd)
    return _sample(key_arr)


# speedup vs baseline: 1.0087x; 1.0046x over previous
import functools

import jax
import jax.numpy as jnp
from jax import lax
from jax.experimental import pallas as pl
from jax.experimental.pallas import tpu as pltpu
from jax.experimental.pallas import tpu_sc as plsc

N_OUT = 16384
SPAN = 1000000

_info = plsc.get_sparse_core_info()
_NC, _NS, _L = _info.num_cores, _info.num_subcores, _info.num_lanes
_NW = _NC * _NS
_PER_W = N_OUT // _NW
_VREGS = _PER_W // _L


def _rotl(x, r):
    return (x << jnp.uint32(r)) | (x >> jnp.uint32(32 - r))


def _threefry2x32(k0, k1, x0, x1):
    ks2 = k0 ^ k1 ^ jnp.uint32(0x1BD11BDA)
    ks = (k0, k1, ks2)
    rots = ((13, 15, 26, 6), (17, 29, 16, 24))
    x0 = x0 + k0
    x1 = x1 + k1
    for i in range(5):
        for r in rots[i % 2]:
            x0 = x0 + x1
            x1 = _rotl(x1, r)
            x1 = x1 ^ x0
        x0 = x0 + ks[(i + 1) % 3]
        x1 = x1 + ks[(i + 2) % 3] + jnp.uint32(i + 1)
    return x0, x1


def _mod_span(bits):
    q = (bits.astype(jnp.float32) * jnp.float32(1.0 / SPAN)).astype(jnp.int32)
    r = (bits - q.astype(jnp.uint32) * jnp.uint32(SPAN)).astype(jnp.int32)
    r = jnp.where(r < 0, r + jnp.int32(SPAN), r)
    r = jnp.where(r >= jnp.int32(SPAN), r - jnp.int32(SPAN), r)
    return r


def _sc_body(key_hbm, out_hbm, key_v, out_v):
    wid = lax.axis_index("s") * _NC + lax.axis_index("c")
    pltpu.sync_copy(key_hbm, key_v)
    kv = key_v[...]
    k0 = jnp.full((_L,), kv[0], jnp.uint32)
    k1 = jnp.full((_L,), kv[1], jnp.uint32)

    zero = jnp.zeros((_L,), jnp.uint32)
    c, d = _threefry2x32(k0, k1, zero, jnp.full((_L,), 1, jnp.uint32))

    base = (wid * _PER_W).astype(jnp.uint32)
    idx = lax.iota(jnp.uint32, _L)
    def step(j, carry):
        cnt = idx + base + (j * _L).astype(jnp.uint32)
        y0, y1 = _threefry2x32(c, d, zero, cnt)
        out_v[pl.ds(j * _L, _L)] = _mod_span(y0 ^ y1)
        return carry

    lax.fori_loop(0, _VREGS, step, 0, unroll=1)

    pltpu.sync_copy(out_v, out_hbm.at[pl.ds(wid * _PER_W, _PER_W)])


@jax.jit
def _sample(key_arr):
    mesh = plsc.VectorSubcoreMesh(
        core_axis_name="c", subcore_axis_name="s", num_cores=_NC)
    f = functools.partial(
        pl.kernel,
        mesh=mesh,
        out_type=jax.ShapeDtypeStruct((N_OUT,), jnp.int32),
        scratch_types=[
            pltpu.VMEM((_L,), jnp.uint32),
            pltpu.VMEM((_PER_W,), jnp.int32),
        ],
    )(_sc_body)
    return f(key_arr)


def kernel(key):
    kd = jax.random.key_data(key).astype(jnp.uint32)
    key_arr = jnp.zeros((_L,), jnp.uint32).at[:2].set(kd)
    return _sample(key_arr)


# speedup vs baseline: 1.0431x; 1.0342x over previous
import functools

import jax
import jax.numpy as jnp
from jax import lax
from jax.experimental import pallas as pl
from jax.experimental.pallas import tpu as pltpu
from jax.experimental.pallas import tpu_sc as plsc

N_OUT = 16384
SPAN = 1000000

_info = plsc.get_sparse_core_info()
_NC, _NS, _L = _info.num_cores, _info.num_subcores, _info.num_lanes
_NW = _NC * _NS
_PER_W = N_OUT // _NW
_VREGS = _PER_W // _L


def _rotl(x, r):
    return (x << jnp.uint32(r)) | (x >> jnp.uint32(32 - r))


def _threefry2x32(k0, k1, x0, x1):
    ks2 = k0 ^ k1 ^ jnp.uint32(0x1BD11BDA)
    ks = (k0, k1, ks2)
    rots = ((13, 15, 26, 6), (17, 29, 16, 24))
    x0 = x0 + k0
    x1 = x1 + k1
    for i in range(5):
        for r in rots[i % 2]:
            x0 = x0 + x1
            x1 = _rotl(x1, r)
            x1 = x1 ^ x0
        x0 = x0 + ks[(i + 1) % 3]
        x1 = x1 + ks[(i + 2) % 3] + jnp.uint32(i + 1)
    return x0, x1


def _mod_span(bits):
    q = (bits.astype(jnp.float32) * jnp.float32(1.0 / SPAN)).astype(jnp.int32)
    r = (bits - q.astype(jnp.uint32) * jnp.uint32(SPAN)).astype(jnp.int32)
    r = jnp.where(r < 0, r + jnp.int32(SPAN), r)
    r = jnp.where(r >= jnp.int32(SPAN), r - jnp.int32(SPAN), r)
    return r


def _sc_body(key_hbm, out_hbm, key_v, out_v):
    wid = lax.axis_index("s") * _NC + lax.axis_index("c")
    pltpu.sync_copy(key_hbm, key_v.at[pl.ds(0, 2)])
    kv = key_v[...]
    k0 = jnp.full((_L,), kv[0], jnp.uint32)
    k1 = jnp.full((_L,), kv[1], jnp.uint32)

    zero = jnp.zeros((_L,), jnp.uint32)
    c, d = _threefry2x32(k0, k1, zero, jnp.full((_L,), 1, jnp.uint32))

    base = (wid * _PER_W).astype(jnp.uint32)
    idx = lax.iota(jnp.uint32, _L)
    def step(j, carry):
        cnt = idx + base + (j * _L).astype(jnp.uint32)
        y0, y1 = _threefry2x32(c, d, zero, cnt)
        out_v[pl.ds(j * _L, _L)] = _mod_span(y0 ^ y1)
        return carry

    lax.fori_loop(0, _VREGS, step, 0, unroll=1)

    pltpu.sync_copy(out_v, out_hbm.at[pl.ds(wid * _PER_W, _PER_W)])


@jax.jit
def _sample(key_arr):
    mesh = plsc.VectorSubcoreMesh(
        core_axis_name="c", subcore_axis_name="s", num_cores=_NC)
    f = functools.partial(
        pl.kernel,
        mesh=mesh,
        out_type=jax.ShapeDtypeStruct((N_OUT,), jnp.int32),
        scratch_types=[
            pltpu.VMEM((_L,), jnp.uint32),
            pltpu.VMEM((_PER_W,), jnp.int32),
        ],
    )(_sc_body)
    return f(key_arr)


def kernel(key):
    return _sample(jax.random.key_data(key).astype(jnp.uint32))
